# baseline (speedup 1.0000x reference)
import functools

import jax
import jax.numpy as jnp
from jax import lax
from jax.experimental import pallas as pl
from jax.experimental.pallas import tpu as pltpu
from jax.experimental.pallas import tpu_sc as plsc

N = 10000
NP = 10240
E = 320000
D_IN = 128
D = 128
NS = 16
EPC = E // 2
EPT = EPC // NS
C = 80
CH = EPT // C
RPT = NP // NS
RS = 8

_f32 = jnp.float32


def _mm_body(x_ref, w_ref, o_ref):
    o_ref[...] = jnp.dot(x_ref[...], w_ref[...], preferred_element_type=_f32)


def _matmul(x, W):
    BM = 2000
    return pl.pallas_call(
        _mm_body,
        grid=(N // BM,),
        in_specs=[
            pl.BlockSpec((BM, D_IN), lambda r: (r, 0)),
            pl.BlockSpec((D_IN, D), lambda r: (0, 0)),
        ],
        out_specs=pl.BlockSpec((BM, D), lambda r: (r, 0)),
        out_shape=jax.ShapeDtypeStruct((NP, D), _f32),
    )(x, W)


def _comb_body(p_ref, o_ref):
    o_ref[...] = p_ref[0] + p_ref[1]


def _combine(P):
    BM = 2000
    return pl.pallas_call(
        _comb_body,
        grid=(N // BM,),
        in_specs=[pl.BlockSpec((2, BM, D), lambda r: (0, r, 0))],
        out_specs=pl.BlockSpec((BM, D), lambda r: (r, 0)),
        out_shape=jax.ShapeDtypeStruct((NP, D), _f32),
    )(P)


def _final_body(p_ref, b_ref, o_ref):
    o_ref[...] = p_ref[0] + p_ref[1] + b_ref[...]


def _final(P, b):
    BM = 2000
    return pl.pallas_call(
        _final_body,
        grid=(N // BM,),
        in_specs=[
            pl.BlockSpec((2, BM, D), lambda r: (0, r, 0)),
            pl.BlockSpec((1, D), lambda r: (0, 0)),
        ],
        out_specs=pl.BlockSpec((BM, D), lambda r: (r, 0)),
        out_shape=jax.ShapeDtypeStruct((N, D), _f32),
    )(P, b.reshape(1, D))


def _sc_body(sup_hbm, erw_hbm, ecol_hbm, p_hbm,
             colm, rb0, rb1, rb2, eb0, eb1, eb2, gb0, gb1, gb2, acc,
             semg0, semg1, semg2, seme0, seme1, seme2, sems0, sems1, sems2):
    c = lax.axis_index("c")
    s = lax.axis_index("s")
    row0 = s * RPT
    zero16 = jnp.zeros((16,), _f32)

    def _zinit(r, _):
        for q in range(D // 16):
            gb0[r, pl.ds(q * 16, 16)] = zero16
        return 0

    lax.fori_loop(0, C, _zinit, 0)
    for j in range(RPT // C):
        pltpu.sync_copy(gb0, acc.at[pl.ds(row0 + j * C, C)])

    t = c * NS + s
    cb = t * CH
    pltpu.sync_copy(ecol_hbm.at[t], colm)
    plsc.subcore_barrier()

    def _gather(g, buf, sem):
        pltpu.async_copy(sup_hbm.at[colm.at[g]], buf, sem)

    def _gdrain(g, buf, sem):
        pltpu.make_async_copy(sup_hbm.at[colm.at[g]], buf, sem).wait()

    def _edges(g, eb, sem):
        pltpu.async_copy(erw_hbm.at[cb + g], eb, sem)

    def _edrain(g, eb, sem):
        pltpu.make_async_copy(erw_hbm.at[cb + g], eb, sem).wait()

    def _compute(gb, eb, rb):
        def grp(j, _):
            base = j * 16
            sl16 = pl.ds(base, 16)
            rb[sl16] = eb[0, sl16].astype(jnp.int32)
            w16 = eb[1, sl16]
            ws = [w16[i] for i in range(16)]
            for i in range(16):
                e = base + i
                for q in range(D // 16):
                    sl = pl.ds(q * 16, 16)
                    gb[e, sl] = gb[e, sl] * ws[i]
            return 0

        lax.fori_loop(0, C // 16, grp, 0)

    def _sca(gb, rb, sem):
        pltpu.async_copy(gb, acc.at[rb], sem, add=True)

    def _sdrain(gb, rb, sem):
        pltpu.make_async_copy(gb, acc.at[rb], sem).wait()

    gbufs = (gb0, gb1, gb2)
    rbufs = (rb0, rb1, rb2)
    ebufs = (eb0, eb1, eb2)
    semgs = (semg0, semg1, semg2)
    semes = (seme0, seme1, seme2)
    semss = (sems0, sems1, sems2)

    def _issue(g, k):
        _edges(g, ebufs[k], semes[k])
        _gather(g, gbufs[k], semgs[k])

    def _process(g, k):
        _gdrain(g, gbufs[k], semgs[k])
        _edrain(g, ebufs[k], semes[k])
        _compute(gbufs[k], ebufs[k], rbufs[k])
        _sca(gbufs[k], rbufs[k], semss[k])

    def _retire(k):
        _sdrain(gbufs[k], rbufs[k], semss[k])

    _issue(0, 0)
    _issue(1, 1)
    _issue(2, 2)
    _process(0, 0)

    def tri_body(p, _):
        g0 = p * 3 + 1
        for k0 in range(3):
            g = g0 + k0
            k = (1 + k0) % 3
            kprev = k0
            _retire(kprev)
            _issue(g + 2, kprev)
            _process(g, k)
        return 0

    lax.fori_loop(0, 40, tri_body, 0)
    _retire(0)
    _issue(123, 0)
    _process(121, 1)
    _retire(1)
    _issue(124, 1)
    _process(122, 2)
    _retire(2)
    _process(123, 0)
    _retire(0)
    _process(124, 1)
    _retire(1)

    plsc.subcore_barrier()
    pltpu.sync_copy(acc.at[pl.ds(row0, RPT)],
                    p_hbm.at[c, pl.ds(row0, RPT)])


@functools.partial(
    pl.kernel,
    out_type=jax.ShapeDtypeStruct((2, NP, D), _f32),
    mesh=plsc.VectorSubcoreMesh(core_axis_name="c", subcore_axis_name="s"),
    scratch_types=[
        pltpu.VMEM((CH, C), jnp.int32),
        pltpu.VMEM((C,), jnp.int32),
        pltpu.VMEM((C,), jnp.int32),
        pltpu.VMEM((C,), jnp.int32),
        pltpu.VMEM((2, C), _f32),
        pltpu.VMEM((2, C), _f32),
        pltpu.VMEM((2, C), _f32),
        pltpu.VMEM((C, D), _f32),
        pltpu.VMEM((C, D), _f32),
        pltpu.VMEM((C, D), _f32),
        pltpu.VMEM_SHARED((NP, D), _f32),
        pltpu.SemaphoreType.DMA,
        pltpu.SemaphoreType.DMA,
        pltpu.SemaphoreType.DMA,
        pltpu.SemaphoreType.DMA,
        pltpu.SemaphoreType.DMA,
        pltpu.SemaphoreType.DMA,
        pltpu.SemaphoreType.DMA,
        pltpu.SemaphoreType.DMA,
        pltpu.SemaphoreType.DMA,
    ],
)
def _sc_spmm(*refs):
    _sc_body(*refs)


def kernel(x, edge_index, edge_weight, W, b):
    rowf = edge_index[0].astype(_f32).reshape(2 * NS * CH, C)
    erw = jnp.stack([rowf, edge_weight.reshape(2 * NS * CH, C)], axis=1)
    col3 = edge_index[1].reshape(2 * NS, CH, C)
    sup = _matmul(x, W)
    sup = _combine(_sc_spmm(sup, erw, col3))
    sup = _combine(_sc_spmm(sup, erw, col3))
    return _final(_sc_spmm(sup, erw, col3), b)


# speedup vs baseline: 1.0485x; 1.0485x over previous
import functools

import jax
import jax.numpy as jnp
from jax import lax
from jax.experimental import pallas as pl
from jax.experimental.pallas import tpu as pltpu
from jax.experimental.pallas import tpu_sc as plsc

N = 10000
NP = 10240
E = 320000
D_IN = 128
D = 128
NS = 16
EPC = E // 2
EPT = EPC // NS
C = 80
CH = EPT // C
RPT = NP // NS
RS = 8

_f32 = jnp.float32


def _mm_body(x_ref, w_ref, o_ref):
    o_ref[...] = jnp.dot(x_ref[...], w_ref[...], preferred_element_type=_f32)


def _matmul(x, W):
    BM = 10000
    return pl.pallas_call(
        _mm_body,
        grid=(N // BM,),
        in_specs=[
            pl.BlockSpec((BM, D_IN), lambda r: (r, 0)),
            pl.BlockSpec((D_IN, D), lambda r: (0, 0)),
        ],
        out_specs=pl.BlockSpec((BM, D), lambda r: (r, 0)),
        out_shape=jax.ShapeDtypeStruct((NP, D), _f32),
    )(x, W)


def _comb_body(p_ref, o_ref):
    o_ref[...] = p_ref[0] + p_ref[1]


def _combine(P):
    BM = 10000
    return pl.pallas_call(
        _comb_body,
        grid=(N // BM,),
        in_specs=[pl.BlockSpec((2, BM, D), lambda r: (0, r, 0))],
        out_specs=pl.BlockSpec((BM, D), lambda r: (r, 0)),
        out_shape=jax.ShapeDtypeStruct((NP, D), _f32),
    )(P)


def _final_body(p_ref, b_ref, o_ref):
    o_ref[...] = p_ref[0] + p_ref[1] + b_ref[...]


def _final(P, b):
    BM = 10000
    return pl.pallas_call(
        _final_body,
        grid=(N // BM,),
        in_specs=[
            pl.BlockSpec((2, BM, D), lambda r: (0, r, 0)),
            pl.BlockSpec((1, D), lambda r: (0, 0)),
        ],
        out_specs=pl.BlockSpec((BM, D), lambda r: (r, 0)),
        out_shape=jax.ShapeDtypeStruct((N, D), _f32),
    )(P, b.reshape(1, D))


def _sc_body(sup_hbm, erow_hbm, ew_hbm, ecol_hbm, p_hbm,
             colm, rb0, rb1, rb2, wb0, wb1, wb2, gb0, gb1, gb2, acc,
             semg0, semg1, semg2, seme0, seme1, seme2, sems0, sems1, sems2):
    c = lax.axis_index("c")
    s = lax.axis_index("s")
    row0 = s * RPT
    zero16 = jnp.zeros((16,), _f32)

    def _zinit(r, _):
        for q in range(D // 16):
            gb0[r, pl.ds(q * 16, 16)] = zero16
        return 0

    lax.fori_loop(0, C, _zinit, 0)
    for j in range(RPT // C):
        pltpu.sync_copy(gb0, acc.at[pl.ds(row0 + j * C, C)])

    t = c * NS + s
    ebase = t * EPT
    pltpu.sync_copy(ecol_hbm.at[t], colm)
    plsc.subcore_barrier()

    def _gather(g, buf, sem):
        pltpu.async_copy(sup_hbm.at[colm.at[g]], buf, sem)

    def _gdrain(g, buf, sem):
        pltpu.make_async_copy(sup_hbm.at[colm.at[g]], buf, sem).wait()

    def _edges(g, rb, wb, sem):
        off = ebase + g * C
        pltpu.async_copy(erow_hbm.at[pl.ds(off, C)], rb, sem)
        pltpu.async_copy(ew_hbm.at[pl.ds(off, C)], wb, sem)

    def _edrain(g, rb, wb, sem):
        off = ebase + g * C
        pltpu.make_async_copy(erow_hbm.at[pl.ds(off, C)], rb, sem).wait()
        pltpu.make_async_copy(ew_hbm.at[pl.ds(off, C)], wb, sem).wait()

    def _compute(gb, wb):
        def grp(j, _):
            base = j * 16
            w16 = wb[pl.ds(base, 16)]
            ws = [w16[i] for i in range(16)]
            for i in range(16):
                e = base + i
                for q in range(D // 16):
                    sl = pl.ds(q * 16, 16)
                    gb[e, sl] = gb[e, sl] * ws[i]
            return 0

        lax.fori_loop(0, C // 16, grp, 0)

    def _sca(gb, rb, sem):
        pltpu.async_copy(gb, acc.at[rb], sem, add=True)

    def _sdrain(gb, rb, sem):
        pltpu.make_async_copy(gb, acc.at[rb], sem).wait()

    gbufs = (gb0, gb1, gb2)
    rbufs = (rb0, rb1, rb2)
    wbufs = (wb0, wb1, wb2)
    semgs = (semg0, semg1, semg2)
    semes = (seme0, seme1, seme2)
    semss = (sems0, sems1, sems2)

    def _issue(g, k):
        _edges(g, rbufs[k], wbufs[k], semes[k])
        _gather(g, gbufs[k], semgs[k])

    def _process(g, k):
        _gdrain(g, gbufs[k], semgs[k])
        _edrain(g, rbufs[k], wbufs[k], semes[k])
        _compute(gbufs[k], wbufs[k])
        _sca(gbufs[k], rbufs[k], semss[k])

    def _retire(k):
        _sdrain(gbufs[k], rbufs[k], semss[k])

    _issue(0, 0)
    _issue(1, 1)
    _issue(2, 2)
    _process(0, 0)

    def tri_body(p, _):
        g0 = p * 3 + 1
        for k0 in range(3):
            g = g0 + k0
            k = (1 + k0) % 3
            kprev = k0
            _retire(kprev)
            _issue(g + 2, kprev)
            _process(g, k)
        return 0

    lax.fori_loop(0, 40, tri_body, 0)
    _retire(0)
    _issue(123, 0)
    _process(121, 1)
    _retire(1)
    _issue(124, 1)
    _process(122, 2)
    _retire(2)
    _process(123, 0)
    _retire(0)
    _process(124, 1)
    _retire(1)

    plsc.subcore_barrier()
    pltpu.sync_copy(acc.at[pl.ds(row0, RPT)],
                    p_hbm.at[c, pl.ds(row0, RPT)])


@functools.partial(
    pl.kernel,
    out_type=jax.ShapeDtypeStruct((2, NP, D), _f32),
    mesh=plsc.VectorSubcoreMesh(core_axis_name="c", subcore_axis_name="s"),
    scratch_types=[
        pltpu.VMEM((CH, C), jnp.int32),
        pltpu.VMEM((C,), jnp.int32),
        pltpu.VMEM((C,), jnp.int32),
        pltpu.VMEM((C,), jnp.int32),
        pltpu.VMEM((C,), _f32),
        pltpu.VMEM((C,), _f32),
        pltpu.VMEM((C,), _f32),
        pltpu.VMEM((C, D), _f32),
        pltpu.VMEM((C, D), _f32),
        pltpu.VMEM((C, D), _f32),
        pltpu.VMEM_SHARED((NP, D), _f32),
        pltpu.SemaphoreType.DMA,
        pltpu.SemaphoreType.DMA,
        pltpu.SemaphoreType.DMA,
        pltpu.SemaphoreType.DMA,
        pltpu.SemaphoreType.DMA,
        pltpu.SemaphoreType.DMA,
        pltpu.SemaphoreType.DMA,
        pltpu.SemaphoreType.DMA,
        pltpu.SemaphoreType.DMA,
    ],
)
def _sc_spmm(*refs):
    _sc_body(*refs)


def kernel(x, edge_index, edge_weight, W, b):
    row = edge_index[0]
    col3 = edge_index[1].reshape(2 * NS, CH, C)
    sup = _matmul(x, W)
    sup = _combine(_sc_spmm(sup, row, edge_weight, col3))
    sup = _combine(_sc_spmm(sup, row, edge_weight, col3))
    return _final(_sc_spmm(sup, row, edge_weight, col3), b)
